# baseline (speedup 1.0000x reference)
import jax
import jax.numpy as jnp
from jax import lax
from jax.experimental import pallas as pl
from jax.experimental.pallas import tpu as pltpu
from jax.experimental.pallas import tpu_sc as plsc

B = 16384
D = 32
NT = 100000
NP = NT // 4
_NC, _NS = 2, 16
_NW = _NC * _NS
_BPW = B // _NW
_CH = 128
_NCH = _BPW // _CH
_IDX_ROWS = B // _CH

_KT = 20
_NCHUNK = NT // (8 * _KT)


def _depad_body(t0, t1, t2, t3, t4, t5, o0, o1, o2, o3, o4, o5,
                ia, ib, oa, ob, sem_i, sem_o):
    wid = lax.axis_index("s") * _NC + lax.axis_index("c")
    nc = jnp.where(wid < _NCHUNK - 32 * (_NCHUNK // 32), (_NCHUNK // 32) + 1,
                   _NCHUNK // 32)

    for tbl, out in ((t0, o0), (t1, o1), (t2, o2), (t3, o3), (t4, o4),
                     (t5, o5)):
        def chunk_fn(j, carry, tbl=tbl, out=out):
            cid = wid + 32 * j
            rbase = pl.multiple_of(cid * (8 * _KT), 8)
            pltpu.async_copy(tbl.at[pl.ds(rbase, 8 * _KT)], ia, sem_i).wait()

            def tile_fn(t, c2):
                for s in range(8):
                    p = 2 * t + (s // 4)
                    lo = (s % 4) * 32
                    oa[p, pl.ds(lo, 16)] = ia[8 * t + s, pl.ds(0, 16)]
                    oa[p, pl.ds(lo + 16, 16)] = ia[8 * t + s, pl.ds(16, 16)]
                return c2

            lax.fori_loop(0, _KT, tile_fn, 0)
            obase = pl.multiple_of(cid * (2 * _KT), 8)
            pltpu.async_copy(oa, out.at[pl.ds(obase, 2 * _KT)],
                             sem_o).wait()
            return carry

        lax.fori_loop(0, nc, chunk_fn, 0)


_sc_depad = pl.kernel(
    _depad_body,
    out_type=[jax.ShapeDtypeStruct((NP, 4 * D), jnp.float32)] * 6,
    mesh=plsc.VectorSubcoreMesh(core_axis_name="c", subcore_axis_name="s"),
    compiler_params=pltpu.CompilerParams(needs_layout_passes=False),
    scratch_types=(
        [pltpu.VMEM((8 * _KT, D), jnp.float32)] * 2
        + [pltpu.VMEM((2 * _KT, 4 * D), jnp.float32)] * 2
        + [pltpu.SemaphoreType.DMA] * 2
    ),
)


def _gather_body(aidx_h, tidx_h, iaidx_h, itidx_h,
                 aeu_h, teu_h, aei_h, tei_h, wa_h, wb_h,
                 aue_o, tue_o, aie_o, tie_o, waa_o, wat_o, wba_o, wbt_o,
                 aidx, tidx, aiidx, tiidx, apidx, tpidx, iapidx, itpidx,
                 ga, gb, oa, ob, sem, sem_o):
    wid = lax.axis_index("s") * _NC + lax.axis_index("c")
    pltpu.sync_copy(aidx_h.at[wid], aidx)
    pltpu.sync_copy(tidx_h.at[wid], tidx)
    pltpu.sync_copy(iaidx_h.at[wid], aiidx)
    pltpu.sync_copy(itidx_h.at[wid], tiidx)

    for src, dst in ((aidx, apidx), (tidx, tpidx), (aiidx, iapidx),
                     (tiidx, itpidx)):
        def shift_fn(m, c2, src=src, dst=dst):
            c = m // 8
            o = (m % 8) * 16
            dst[c, pl.ds(o, 16)] = lax.shift_right_logical(
                src[c, pl.ds(o, 16)], 2)
            return c2

        lax.fori_loop(0, _NCH * 8, shift_fn, 0)

    jobs = [(aeu_h, apidx, aidx, aue_o), (teu_h, tpidx, tidx, tue_o),
            (aei_h, iapidx, aiidx, aie_o), (tei_h, itpidx, tiidx, tie_o),
            (wa_h, apidx, aidx, waa_o), (wa_h, tpidx, tidx, wat_o),
            (wb_h, apidx, aidx, wba_o), (wb_h, tpidx, tidx, wbt_o)]

    lanes16 = lax.iota(jnp.int32, 16)

    def extract(gbuf, obuf, idx, c):
        def group_fn(m, c2):
            ivec = 16 * m + lanes16
            rv = idx[c, pl.ds(16 * m, 16)]
            offv = lax.shift_left(lax.bitwise_and(rv, 3), 5)
            prow = lax.shift_right_logical(ivec, 2)
            pcol = lax.shift_left(lax.bitwise_and(ivec, 3), 5)
            for cc in range(32):
                v = plsc.load_gather(gbuf, [ivec, offv + cc])
                plsc.store_scatter(obuf, [prow, pcol + cc], v)
            return c2

        lax.fori_loop(0, 8, group_fn, 0)

    for tbl, pidx, idx, out in jobs:
        prev = pltpu.async_copy(tbl.at[pidx.at[0]], ga, sem)
        for c in range(_NCH):
            cur = None
            if c + 1 < _NCH:
                cur = pltpu.async_copy(
                    tbl.at[pidx.at[c + 1]], gb if c % 2 == 0 else ga, sem)
            prev.wait()
            gbuf = ga if c % 2 == 0 else gb
            obuf = oa if c % 2 == 0 else ob
            extract(gbuf, obuf, idx, c)
            pltpu.sync_copy(
                obuf,
                out.at[pl.ds(pl.multiple_of(wid * _CH + c * (_CH // 4), 8),
                             _CH // 4)])
            prev = cur


_sc_gather = pl.kernel(
    _gather_body,
    out_type=[jax.ShapeDtypeStruct((B // 4, 4 * D), jnp.float32)] * 8,
    mesh=plsc.VectorSubcoreMesh(core_axis_name="c", subcore_axis_name="s"),
    compiler_params=pltpu.CompilerParams(needs_layout_passes=False),
    scratch_types=(
        [pltpu.VMEM((_NCH, _CH), jnp.int32)] * 8
        + [pltpu.VMEM((_CH, 4 * D), jnp.float32)] * 2
        + [pltpu.VMEM((_CH // 4, 4 * D), jnp.float32)] * 2
        + [pltpu.SemaphoreType.DMA] * 2
    ),
)

_BT = 512
_GRID = (B // 4) // _BT
_PK = B // 4


def _tc_body(ar_ref, tr_ref,
             aue, tue, aie, tie, waa, wat, wba, wbt,
             w1au, b1au, w2au, b2au,
             w1tu, b1tu, w2tu, b2tu,
             w1ai, b1ai, w2ai, b2ai,
             w1ti, b1ti, w2ti, b2ti,
             sel, la_ref, lt_ref,
             w1s0, w1s1, w1s2, w1s3, w2s0, w2s1, w2s2, w2s3):
    i = pl.program_id(0)

    @pl.when(i == 0)
    def _():
        for ws, w, d_in, d_out in (
                (w1s0, w1au, D, 2 * D), (w1s1, w1tu, D, 2 * D),
                (w1s2, w1ai, D, 2 * D), (w1s3, w1ti, D, 2 * D),
                (w2s0, w2au, 2 * D, D), (w2s1, w2tu, 2 * D, D),
                (w2s2, w2ai, 2 * D, D), (w2s3, w2ti, 2 * D, D)):
            ws[...] = jnp.zeros((4 * d_in, 4 * d_out), jnp.float32)
            for k in range(4):
                ws[k * d_in:(k + 1) * d_in, k * d_out:(k + 1) * d_out] = w[...]

    a_e = aue[...]
    t_e = tue[...]
    x_au = waa[...] * a_e + (1.0 - wat[...]) * t_e
    x_tu = wba[...] * a_e + (1.0 - wbt[...]) * t_e
    x_ai = aie[...]
    x_ti = tie[...]

    def mlp(x, w1, b1, w2, b2):
        b1t = jnp.concatenate([b1[...]] * 4, axis=1)
        b2t = jnp.concatenate([b2[...]] * 4, axis=1)
        h = jnp.maximum(
            jnp.dot(x, w1[...], preferred_element_type=jnp.float32)
            + b1t, 0.0)
        return jnp.maximum(
            jnp.dot(h, w2[...], preferred_element_type=jnp.float32)
            + b2t, 0.0)

    y_au = mlp(x_au, w1s0, b1au, w2s0, b2au)
    y_tu = mlp(x_tu, w1s1, b1tu, w2s1, b2tu)
    y_ai = mlp(x_ai, w1s2, b1ai, w2s2, b2ai)
    y_ti = mlp(x_ti, w1s3, b1ti, w2s3, b2ti)

    s_a = jnp.maximum(
        jnp.dot(y_au * y_ai, sel[...], preferred_element_type=jnp.float32),
        1e-6)
    s_t = jnp.maximum(
        jnp.dot(y_tu * y_ti, sel[...], preferred_element_type=jnp.float32),
        1e-6)
    da = s_a - ar_ref[...]
    dt = s_t - tr_ref[...]
    pa = jnp.sum(da * da) * (1.0 / B)
    pt = jnp.sum(dt * dt) * (1.0 / B)

    @pl.when(i == 0)
    def _():
        la_ref[0, 0] = 0.0
        lt_ref[0, 0] = 0.0

    la_ref[0, 0] += pa
    lt_ref[0, 0] += pt


def _wspec():
    return pl.BlockSpec((D, 2 * D), lambda i: (0, 0))


def _bspec():
    return pl.BlockSpec((1, 2 * D), lambda i: (0, 0))


def _w2spec():
    return pl.BlockSpec((2 * D, D), lambda i: (0, 0))


def _b2spec():
    return pl.BlockSpec((1, D), lambda i: (0, 0))


_tc_dense = pl.pallas_call(
    _tc_body,
    grid=(_GRID,),
    in_specs=[
        pl.BlockSpec((_BT, 4), lambda i: (i, 0)),
        pl.BlockSpec((_BT, 4), lambda i: (i, 0)),
    ] + [pl.BlockSpec((_BT, 4 * D), lambda i: (i, 0))] * 8 + [
        _wspec(), _bspec(), _w2spec(), _b2spec(),
        _wspec(), _bspec(), _w2spec(), _b2spec(),
        _wspec(), _bspec(), _w2spec(), _b2spec(),
        _wspec(), _bspec(), _w2spec(), _b2spec(),
        pl.BlockSpec((4 * D, 4), lambda i: (0, 0)),
    ],
    out_specs=[
        pl.BlockSpec(memory_space=pltpu.SMEM),
        pl.BlockSpec(memory_space=pltpu.SMEM),
    ],
    out_shape=[jax.ShapeDtypeStruct((1, 1), jnp.float32)] * 2,
    scratch_shapes=(
        [pltpu.VMEM((4 * D, 8 * D), jnp.float32)] * 4
        + [pltpu.VMEM((8 * D, 4 * D), jnp.float32)] * 4
    ),
)


def kernel(ausers, aitems, aratings, tusers, titems, tratings, params):
    p = params
    packed_tables = _sc_depad(
        p["a_emb_user"], p["t_emb_user"], p["a_emb_item"], p["t_emb_item"],
        p["W_a"], p["W_b"])

    au2 = ausers.reshape(_NW, _NCH, _CH)
    tu2 = tusers.reshape(_NW, _NCH, _CH)
    ai2 = aitems.reshape(_NW, _NCH, _CH)
    ti2 = titems.reshape(_NW, _NCH, _CH)
    packed = _sc_gather(au2, tu2, ai2, ti2, *packed_tables)

    wargs = []
    for name in ("mlp_a_users", "mlp_t_users", "mlp_a_items", "mlp_t_items"):
        m = p[name]
        wargs += [
            m["W1"],
            m["b1"].reshape(1, 2 * D),
            m["W2"],
            m["b2"].reshape(1, D),
        ]
    sel = (jnp.arange(4 * D)[:, None] // D ==
           jnp.arange(4)[None, :]).astype(jnp.float32)

    ar2 = aratings.astype(jnp.float32).reshape(_PK, 4)
    tr2 = tratings.astype(jnp.float32).reshape(_PK, 4)
    la, lt = _tc_dense(ar2, tr2, *packed, *wargs, sel)
    return (la[0, 0], lt[0, 0])


# speedup vs baseline: 2.5349x; 2.5349x over previous
import jax
import jax.numpy as jnp
from jax import lax
from jax.experimental import pallas as pl
from jax.experimental.pallas import tpu as pltpu
from jax.experimental.pallas import tpu_sc as plsc

B = 16384
D = 32
NT = 100000
_NC, _NS = 2, 16
_NW = _NC * _NS
_BPW = B // _NW
_CH = 128
_NCH = _BPW // _CH
_IDX_ROWS = B // _CH


def _mk_gather(nsets):
    def body(*args):
        idx_h = args[0:nsets]
        tbl = args[nsets]
        outs = args[nsets + 1:2 * nsets + 1]
        scratch = args[2 * nsets + 1:]
        idxv = scratch[0:nsets]
        bufs = scratch[nsets:3 * nsets]
        sem = scratch[-1]
        wid = lax.axis_index("s") * _NC + lax.axis_index("c")
        rbase = wid * _BPW
        ibase = wid * _NCH
        for k in range(nsets):
            pltpu.sync_copy(idx_h[k].at[pl.ds(ibase, _NCH)], idxv[k])

        def fire(c):
            return [pltpu.async_copy(tbl.at[idxv[k].at[c]],
                                     bufs[2 * k + c % 2], sem)
                    for k in range(nsets)]

        def copy_out(c):
            for k in range(nsets):
                pltpu.sync_copy(bufs[2 * k + c % 2],
                                outs[k].at[pl.ds(rbase + c * _CH, _CH)])

        prev = fire(0)
        for c in range(1, _NCH):
            cur = fire(c)
            for d in prev:
                d.wait()
            copy_out(c - 1)
            prev = cur
        for d in prev:
            d.wait()
        copy_out(_NCH - 1)

    return pl.kernel(
        body,
        out_type=[jax.ShapeDtypeStruct((B, D), jnp.float32)] * nsets,
        mesh=plsc.VectorSubcoreMesh(core_axis_name="c", subcore_axis_name="s"),
        compiler_params=pltpu.CompilerParams(use_tc_tiling_on_sc=False),
        scratch_types=(
            [pltpu.VMEM((_NCH, _CH), jnp.int32)] * nsets
            + [pltpu.VMEM((_CH, D), jnp.float32)] * (2 * nsets)
            + [pltpu.SemaphoreType.DMA]
        ),
    )


_g1 = _mk_gather(1)
_g2 = _mk_gather(2)

_BT = 512
_GRID = (B // 4) // _BT
_PK = B // 4


def _tc_body(ar_ref, tr_ref,
             aue, tue, aie, tie, waa, wat, wba, wbt,
             w1au, b1au, w2au, b2au,
             w1tu, b1tu, w2tu, b2tu,
             w1ai, b1ai, w2ai, b2ai,
             w1ti, b1ti, w2ti, b2ti,
             sel, la_ref, lt_ref,
             w1s0, w1s1, w1s2, w1s3, w2s0, w2s1, w2s2, w2s3):
    i = pl.program_id(0)

    @pl.when(i == 0)
    def _():
        for ws, w, d_in, d_out in (
                (w1s0, w1au, D, 2 * D), (w1s1, w1tu, D, 2 * D),
                (w1s2, w1ai, D, 2 * D), (w1s3, w1ti, D, 2 * D),
                (w2s0, w2au, 2 * D, D), (w2s1, w2tu, 2 * D, D),
                (w2s2, w2ai, 2 * D, D), (w2s3, w2ti, 2 * D, D)):
            ws[...] = jnp.zeros((4 * d_in, 4 * d_out), jnp.float32)
            for k in range(4):
                ws[k * d_in:(k + 1) * d_in, k * d_out:(k + 1) * d_out] = w[...]
    a_e = aue[...].astype(jnp.float32)
    t_e = tue[...].astype(jnp.float32)
    x_au = waa[...].astype(jnp.float32) * a_e + \
        (1.0 - wat[...].astype(jnp.float32)) * t_e
    x_tu = wba[...].astype(jnp.float32) * a_e + \
        (1.0 - wbt[...].astype(jnp.float32)) * t_e
    x_ai = aie[...].astype(jnp.float32)
    x_ti = tie[...].astype(jnp.float32)

    def mlp(x, w1, b1, w2, b2):
        b1t = jnp.concatenate([b1[...]] * 4, axis=1)
        b2t = jnp.concatenate([b2[...]] * 4, axis=1)
        h = jnp.maximum(
            jnp.dot(x, w1[...], preferred_element_type=jnp.float32)
            + b1t, 0.0)
        return jnp.maximum(
            jnp.dot(h, w2[...], preferred_element_type=jnp.float32)
            + b2t, 0.0)

    y_au = mlp(x_au, w1s0, b1au, w2s0, b2au)
    y_tu = mlp(x_tu, w1s1, b1tu, w2s1, b2tu)
    y_ai = mlp(x_ai, w1s2, b1ai, w2s2, b2ai)
    y_ti = mlp(x_ti, w1s3, b1ti, w2s3, b2ti)

    s_a = jnp.maximum(
        jnp.dot(y_au * y_ai, sel[...], preferred_element_type=jnp.float32),
        1e-6)
    s_t = jnp.maximum(
        jnp.dot(y_tu * y_ti, sel[...], preferred_element_type=jnp.float32),
        1e-6)
    da = s_a - ar_ref[...]
    dt = s_t - tr_ref[...]
    pa = jnp.sum(da * da) * (1.0 / B)
    pt = jnp.sum(dt * dt) * (1.0 / B)

    @pl.when(i == 0)
    def _():
        la_ref[0, 0] = 0.0
        lt_ref[0, 0] = 0.0

    la_ref[0, 0] += pa
    lt_ref[0, 0] += pt


def _wspec():
    return pl.BlockSpec((D, 2 * D), lambda i: (0, 0))


def _bspec():
    return pl.BlockSpec((1, 2 * D), lambda i: (0, 0))


def _w2spec():
    return pl.BlockSpec((2 * D, D), lambda i: (0, 0))


def _b2spec():
    return pl.BlockSpec((1, D), lambda i: (0, 0))


_tc_dense = pl.pallas_call(
    _tc_body,
    grid=(_GRID,),
    in_specs=[
        pl.BlockSpec((_BT, 4), lambda i: (i, 0)),
        pl.BlockSpec((_BT, 4), lambda i: (i, 0)),
    ] + [pl.BlockSpec((_BT, 4 * D), lambda i: (i, 0))] * 8 + [
        _wspec(), _bspec(), _w2spec(), _b2spec(),
        _wspec(), _bspec(), _w2spec(), _b2spec(),
        _wspec(), _bspec(), _w2spec(), _b2spec(),
        _wspec(), _bspec(), _w2spec(), _b2spec(),
        pl.BlockSpec((4 * D, 4), lambda i: (0, 0)),
    ],
    out_specs=[
        pl.BlockSpec(memory_space=pltpu.SMEM),
        pl.BlockSpec(memory_space=pltpu.SMEM),
    ],
    out_shape=[jax.ShapeDtypeStruct((1, 1), jnp.float32)] * 2,
    scratch_shapes=(
        [pltpu.VMEM((4 * D, 8 * D), jnp.float32)] * 4
        + [pltpu.VMEM((8 * D, 4 * D), jnp.float32)] * 4
    ),
)


def kernel(ausers, aitems, aratings, tusers, titems, tratings, params):
    p = params
    au2 = ausers.reshape(_IDX_ROWS, _CH)
    tu2 = tusers.reshape(_IDX_ROWS, _CH)
    ai2 = aitems.reshape(_IDX_ROWS, _CH)
    ti2 = titems.reshape(_IDX_ROWS, _CH)
    (aue_g,) = _g1(au2, p["a_emb_user"])
    (tue_g,) = _g1(tu2, p["t_emb_user"])
    (aie_g,) = _g1(ai2, p["a_emb_item"])
    (tie_g,) = _g1(ti2, p["t_emb_item"])
    waa_g, wat_g = _g2(au2, tu2, p["W_a"])
    wba_g, wbt_g = _g2(au2, tu2, p["W_b"])
    gathered = [aue_g, tue_g, aie_g, tie_g, waa_g, wat_g, wba_g, wbt_g]
    packed = [g.reshape(_PK, 4 * D) for g in gathered]

    wargs = []
    for name in ("mlp_a_users", "mlp_t_users", "mlp_a_items", "mlp_t_items"):
        m = p[name]
        wargs += [
            m["W1"],
            m["b1"].reshape(1, 2 * D),
            m["W2"],
            m["b2"].reshape(1, D),
        ]
    sel = (jnp.arange(4 * D)[:, None] // D ==
           jnp.arange(4)[None, :]).astype(jnp.float32)

    ar2 = aratings.astype(jnp.float32).reshape(_PK, 4)
    tr2 = tratings.astype(jnp.float32).reshape(_PK, 4)
    la, lt = _tc_dense(ar2, tr2, *packed, *wargs, sel)
    return (la[0, 0], lt[0, 0])


# speedup vs baseline: 2.5504x; 1.0062x over previous
import jax
import jax.numpy as jnp
from jax import lax
from jax.experimental import pallas as pl
from jax.experimental.pallas import tpu as pltpu
from jax.experimental.pallas import tpu_sc as plsc

B = 16384
D = 32
NT = 100000
_NC, _NS = 2, 16
_NW = _NC * _NS
_BPW = B // _NW
_CH = 128
_NCH = _BPW // _CH
_IDX_ROWS = B // _CH


def _mk_gather(nsets):
    def body(*args):
        idx_h = args[0:nsets]
        tbl = args[nsets]
        outs = args[nsets + 1:2 * nsets + 1]
        scratch = args[2 * nsets + 1:]
        idxv = scratch[0:nsets]
        bufs = scratch[nsets:3 * nsets]
        sem = scratch[-1]
        wid = lax.axis_index("s") * _NC + lax.axis_index("c")
        rbase = wid * _BPW
        ibase = wid * _NCH
        for k in range(nsets):
            pltpu.sync_copy(idx_h[k].at[pl.ds(ibase, _NCH)], idxv[k])

        def fire(c):
            return [pltpu.async_copy(tbl.at[idxv[k].at[c]],
                                     bufs[2 * k + c % 2], sem)
                    for k in range(nsets)]

        def copy_out(c):
            for k in range(nsets):
                pltpu.sync_copy(bufs[2 * k + c % 2],
                                outs[k].at[pl.ds(rbase + c * _CH, _CH)])

        prev = fire(0)
        for c in range(1, _NCH):
            cur = fire(c)
            for d in prev:
                d.wait()
            copy_out(c - 1)
            prev = cur
        for d in prev:
            d.wait()
        copy_out(_NCH - 1)

    return pl.kernel(
        body,
        out_type=[jax.ShapeDtypeStruct((B, D), jnp.float32)] * nsets,
        mesh=plsc.VectorSubcoreMesh(core_axis_name="c", subcore_axis_name="s"),
        compiler_params=pltpu.CompilerParams(use_tc_tiling_on_sc=False),
        scratch_types=(
            [pltpu.VMEM((_NCH, _CH), jnp.int32)] * nsets
            + [pltpu.VMEM((_CH, D), jnp.float32)] * (2 * nsets)
            + [pltpu.SemaphoreType.DMA]
        ),
    )


_g1 = _mk_gather(1)
_g2 = _mk_gather(2)

_BT = 1024
_GRID = (B // 4) // _BT
_PK = B // 4


def _tc_body(ar_ref, tr_ref,
             aue, tue, aie, tie, waa, wat, wba, wbt,
             w1au, b1au, w2au, b2au,
             w1tu, b1tu, w2tu, b2tu,
             w1ai, b1ai, w2ai, b2ai,
             w1ti, b1ti, w2ti, b2ti,
             sel, la_ref, lt_ref,
             w1s0, w1s1, w1s2, w1s3, w2s0, w2s1, w2s2, w2s3):
    i = pl.program_id(0)

    @pl.when(i == 0)
    def _():
        for ws, w, d_in, d_out in (
                (w1s0, w1au, D, 2 * D), (w1s1, w1tu, D, 2 * D),
                (w1s2, w1ai, D, 2 * D), (w1s3, w1ti, D, 2 * D),
                (w2s0, w2au, 2 * D, D), (w2s1, w2tu, 2 * D, D),
                (w2s2, w2ai, 2 * D, D), (w2s3, w2ti, 2 * D, D)):
            ws[...] = jnp.zeros((4 * d_in, 4 * d_out), jnp.float32)
            for k in range(4):
                ws[k * d_in:(k + 1) * d_in, k * d_out:(k + 1) * d_out] = w[...]
    a_e = aue[...].astype(jnp.float32)
    t_e = tue[...].astype(jnp.float32)
    x_au = waa[...].astype(jnp.float32) * a_e + \
        (1.0 - wat[...].astype(jnp.float32)) * t_e
    x_tu = wba[...].astype(jnp.float32) * a_e + \
        (1.0 - wbt[...].astype(jnp.float32)) * t_e
    x_ai = aie[...].astype(jnp.float32)
    x_ti = tie[...].astype(jnp.float32)

    def mlp(x, w1, b1, w2, b2):
        b1t = jnp.concatenate([b1[...]] * 4, axis=1)
        b2t = jnp.concatenate([b2[...]] * 4, axis=1)
        h = jnp.maximum(
            jnp.dot(x, w1[...], preferred_element_type=jnp.float32)
            + b1t, 0.0)
        return jnp.maximum(
            jnp.dot(h, w2[...], preferred_element_type=jnp.float32)
            + b2t, 0.0)

    y_au = mlp(x_au, w1s0, b1au, w2s0, b2au)
    y_tu = mlp(x_tu, w1s1, b1tu, w2s1, b2tu)
    y_ai = mlp(x_ai, w1s2, b1ai, w2s2, b2ai)
    y_ti = mlp(x_ti, w1s3, b1ti, w2s3, b2ti)

    s_a = jnp.maximum(
        jnp.dot(y_au * y_ai, sel[...], preferred_element_type=jnp.float32),
        1e-6)
    s_t = jnp.maximum(
        jnp.dot(y_tu * y_ti, sel[...], preferred_element_type=jnp.float32),
        1e-6)
    da = s_a - ar_ref[...].astype(jnp.float32)
    dt = s_t - tr_ref[...].astype(jnp.float32)
    pa = jnp.sum(da * da) * (1.0 / B)
    pt = jnp.sum(dt * dt) * (1.0 / B)

    @pl.when(i == 0)
    def _():
        la_ref[0, 0] = 0.0
        lt_ref[0, 0] = 0.0

    la_ref[0, 0] += pa
    lt_ref[0, 0] += pt


def _wspec():
    return pl.BlockSpec((D, 2 * D), lambda i: (0, 0))


def _bspec():
    return pl.BlockSpec((1, 2 * D), lambda i: (0, 0))


def _w2spec():
    return pl.BlockSpec((2 * D, D), lambda i: (0, 0))


def _b2spec():
    return pl.BlockSpec((1, D), lambda i: (0, 0))


_tc_dense = pl.pallas_call(
    _tc_body,
    grid=(_GRID,),
    in_specs=[
        pl.BlockSpec((_BT, 4), lambda i: (i, 0)),
        pl.BlockSpec((_BT, 4), lambda i: (i, 0)),
    ] + [pl.BlockSpec((_BT, 4 * D), lambda i: (i, 0))] * 8 + [
        _wspec(), _bspec(), _w2spec(), _b2spec(),
        _wspec(), _bspec(), _w2spec(), _b2spec(),
        _wspec(), _bspec(), _w2spec(), _b2spec(),
        _wspec(), _bspec(), _w2spec(), _b2spec(),
        pl.BlockSpec((4 * D, 4), lambda i: (0, 0)),
    ],
    out_specs=[
        pl.BlockSpec(memory_space=pltpu.SMEM),
        pl.BlockSpec(memory_space=pltpu.SMEM),
    ],
    out_shape=[jax.ShapeDtypeStruct((1, 1), jnp.float32)] * 2,
    scratch_shapes=(
        [pltpu.VMEM((4 * D, 8 * D), jnp.float32)] * 4
        + [pltpu.VMEM((8 * D, 4 * D), jnp.float32)] * 4
    ),
)


def kernel(ausers, aitems, aratings, tusers, titems, tratings, params):
    p = params
    au2 = ausers.reshape(_IDX_ROWS, _CH)
    tu2 = tusers.reshape(_IDX_ROWS, _CH)
    ai2 = aitems.reshape(_IDX_ROWS, _CH)
    ti2 = titems.reshape(_IDX_ROWS, _CH)
    (aue_g,) = _g1(au2, p["a_emb_user"])
    (tue_g,) = _g1(tu2, p["t_emb_user"])
    (aie_g,) = _g1(ai2, p["a_emb_item"])
    (tie_g,) = _g1(ti2, p["t_emb_item"])
    waa_g, wat_g = _g2(au2, tu2, p["W_a"])
    wba_g, wbt_g = _g2(au2, tu2, p["W_b"])
    gathered = [aue_g, tue_g, aie_g, tie_g, waa_g, wat_g, wba_g, wbt_g]
    packed = [g.reshape(_PK, 4 * D) for g in gathered]

    wargs = []
    for name in ("mlp_a_users", "mlp_t_users", "mlp_a_items", "mlp_t_items"):
        m = p[name]
        wargs += [
            m["W1"],
            m["b1"].reshape(1, 2 * D),
            m["W2"],
            m["b2"].reshape(1, D),
        ]
    sel = (jnp.arange(4 * D)[:, None] // D ==
           jnp.arange(4)[None, :]).astype(jnp.float32)

    ar2 = aratings.reshape(_PK, 4)
    tr2 = tratings.reshape(_PK, 4)
    la, lt = _tc_dense(ar2, tr2, *packed, *wargs, sel)
    return (la[0, 0], lt[0, 0])
